# baseline (speedup 1.0000x reference)
import functools

import jax
import jax.numpy as jnp
from jax import lax
from jax.experimental import pallas as pl
from jax.experimental.pallas import tpu as pltpu
from jax.experimental.pallas import tpu_sc as plsc

N = 10000
E = 160000
NF = 8
EF = 4
DIM = 64
B = 64
W = 128

N_PAD = 10240
E_PAD = 163840
NW = 32
TE = E_PAD // NW
CH = 512
NCHUNK = TE // CH
IR = CH // 128

_mesh = functools.partial(
    plsc.VectorSubcoreMesh, core_axis_name="c", subcore_axis_name="s")


def _leaky(v):
    return jnp.where(v >= 0, v, 0.01 * v)


def _sc_gather(h, idx_r):
    @functools.partial(
        pl.kernel, mesh=_mesh(),
        out_type=jax.ShapeDtypeStruct((E_PAD, W), jnp.float32),
        scratch_types=[
            pltpu.VMEM((IR, 128), jnp.int32),
            pltpu.VMEM((CH, W), jnp.float32),
            pltpu.SemaphoreType.DMA,
        ],
    )
    def k(h_hbm, idx_hbm, out_hbm, idx_v, rows_v, sem):
        wid = lax.axis_index("s") * 2 + lax.axis_index("c")
        for c in range(NCHUNK):
            pltpu.sync_copy(idx_hbm.at[wid, c], idx_v)
            for j in range(IR):
                pltpu.async_copy(
                    h_hbm.at[idx_v.at[j]],
                    rows_v.at[pl.ds(j * 128, 128)], sem).wait()
            pltpu.sync_copy(rows_v, out_hbm.at[pl.ds(wid * TE + c * CH, CH)])

    return k(h, idx_r)


_NH = N_PAD // 2
_AR = 5248
_TRASH = 5200
_CPT = E_PAD // 16
_NC2 = _CPT // CH


def _sc_scatter(msg, dst_r2, zinit):
    @functools.partial(
        pl.kernel, mesh=_mesh(),
        out_type=jax.ShapeDtypeStruct((N_PAD, W), jnp.float32),
        scratch_types=[
            pltpu.VMEM((IR, 128), jnp.int32),
            pltpu.VMEM((CH, W), jnp.float32),
            pltpu.VMEM_SHARED((_AR, W), jnp.float32),
        ],
    )
    def k(msg_hbm, dst_hbm, z_hbm, out_hbm, idx_v, rows_v, acc_sh):
        cid = lax.axis_index("c")
        sid = lax.axis_index("s")
        lo = cid * _NH
        pltpu.sync_copy(z_hbm.at[pl.ds(sid * 328, 328)],
                        acc_sh.at[pl.ds(sid * 328, 328)])
        plsc.subcore_barrier()
        for c in range(_NC2):
            pltpu.sync_copy(dst_hbm.at[sid, c], idx_v)
            pltpu.sync_copy(
                msg_hbm.at[pl.ds(sid * _CPT + c * CH, CH)], rows_v)
            for r in range(IR):
                for v in range(8):
                    d = idx_v[r, pl.ds(v * 16, 16)] - lo
                    ok = (d >= 0) & (d < _NH)
                    idx_v[r, pl.ds(v * 16, 16)] = jnp.where(ok, d, _TRASH)
            for r in range(IR):
                pltpu.sync_copy(rows_v.at[pl.ds(r * 128, 128)],
                                acc_sh.at[idx_v.at[r]], add=True)
        plsc.subcore_barrier()
        pltpu.sync_copy(acc_sh.at[pl.ds(sid * 320, 320)],
                        out_hbm.at[pl.ds(cid * _NH + sid * 320, 320)])

    return k(msg, dst_r2, zinit)


_BE = 512


def _tc_prep_wk(edge_attr_p, We1T, be1r, We2T, be2r):
    def body(ea_ref, w1_ref, b1_ref, w2_ref, b2_ref, out_ref):
        h1 = _leaky(
            jnp.dot(ea_ref[...], w1_ref[...],
                    preferred_element_type=jnp.float32) + b1_ref[...])
        wkf = jnp.dot(
            h1, w2_ref[...], preferred_element_type=jnp.float32) + b2_ref[...]
        out_ref[...] = wkf.astype(jnp.bfloat16)

    return pl.pallas_call(
        body,
        grid=(E_PAD // _BE,),
        in_specs=[
            pl.BlockSpec((_BE, EF), lambda i: (i, 0)),
            pl.BlockSpec((EF, DIM), lambda i: (0, 0)),
            pl.BlockSpec((1, DIM), lambda i: (0, 0)),
            pl.BlockSpec((DIM, DIM * DIM), lambda i: (0, 0)),
            pl.BlockSpec((1, DIM * DIM), lambda i: (0, 0)),
        ],
        out_specs=pl.BlockSpec((_BE, DIM * DIM), lambda i: (i, 0)),
        out_shape=jax.ShapeDtypeStruct((E_PAD, DIM * DIM), jnp.bfloat16),
    )(edge_attr_p, We1T, be1r, We2T, be2r)


def _tc_msg(xj, wk, S, R):
    def body(xj_ref, wk_ref, s_ref, r_ref, out_ref):
        x = xj_ref[:, 0:DIM]
        xrep = jnp.dot(x, s_ref[...], preferred_element_type=jnp.float32,
                       precision=lax.Precision.HIGHEST)
        g = xrep * wk_ref[...].astype(jnp.float32)
        acc = jnp.dot(g, r_ref[...], preferred_element_type=jnp.float32,
                      precision=lax.Precision.HIGHEST)
        one = (lax.broadcasted_iota(jnp.int32, (_BE, DIM), 1) == 0
               ).astype(jnp.float32)
        out_ref[...] = jnp.concatenate([acc, one], axis=1)

    return pl.pallas_call(
        body,
        grid=(E_PAD // _BE,),
        in_specs=[
            pl.BlockSpec((_BE, W), lambda i: (i, 0)),
            pl.BlockSpec((_BE, DIM * DIM), lambda i: (i, 0)),
            pl.BlockSpec((DIM, DIM * DIM), lambda i: (0, 0)),
            pl.BlockSpec((DIM * DIM, DIM), lambda i: (0, 0)),
        ],
        out_specs=pl.BlockSpec((_BE, W), lambda i: (i, 0)),
        out_shape=jax.ShapeDtypeStruct((E_PAD, W), jnp.float32),
    )(xj, wk, S, R)


def _tc_h0(x_p, W0T, b0r):
    def body(x_ref, w_ref, b_ref, out_ref):
        h0 = _leaky(
            jnp.dot(x_ref[...], w_ref[...],
                    preferred_element_type=jnp.float32) + b_ref[...])
        out_ref[...] = jnp.concatenate(
            [h0, jnp.zeros((N_PAD, W - DIM), jnp.float32)], axis=1)

    return pl.pallas_call(
        body,
        grid=(1,),
        in_specs=[
            pl.BlockSpec((N_PAD, NF), lambda i: (0, 0)),
            pl.BlockSpec((NF, DIM), lambda i: (0, 0)),
            pl.BlockSpec((1, DIM), lambda i: (0, 0)),
        ],
        out_specs=pl.BlockSpec((N_PAD, W), lambda i: (0, 0)),
        out_shape=jax.ShapeDtypeStruct((N_PAD, W), jnp.float32),
    )(x_p, W0T, b0r)


_BN = 2048


def _tc_gru(h, aggp, WrootT, bconvr, Wg, bg):
    def body(h_ref, ap_ref, wr_ref, bc_ref, wg_ref, bg_ref, out_ref):
        h_ = h_ref[:, 0:DIM]
        deg = jnp.maximum(ap_ref[:, DIM:DIM + 1], 1.0)
        agg = ap_ref[:, 0:DIM] / deg
        m = _leaky(agg + jnp.dot(h_, wr_ref[...],
                                 preferred_element_type=jnp.float32)
                   + bc_ref[...])
        gi = jnp.dot(m, wg_ref[:, 0:3 * DIM],
                     preferred_element_type=jnp.float32) + bg_ref[:, 0:3 * DIM]
        gh = jnp.dot(h_, wg_ref[:, 3 * DIM:6 * DIM],
                     preferred_element_type=jnp.float32) + bg_ref[:, 3 * DIM:]
        r = jax.nn.sigmoid(gi[:, 0:DIM] + gh[:, 0:DIM])
        z = jax.nn.sigmoid(gi[:, DIM:2 * DIM] + gh[:, DIM:2 * DIM])
        n = jnp.tanh(gi[:, 2 * DIM:3 * DIM] + r * gh[:, 2 * DIM:3 * DIM])
        hn = (1.0 - z) * n + z * h_
        out_ref[...] = jnp.concatenate(
            [hn, jnp.zeros((_BN, W - DIM), jnp.float32)], axis=1)

    return pl.pallas_call(
        body,
        grid=(N_PAD // _BN,),
        in_specs=[
            pl.BlockSpec((_BN, W), lambda i: (i, 0)),
            pl.BlockSpec((_BN, W), lambda i: (i, 0)),
            pl.BlockSpec((DIM, DIM), lambda i: (0, 0)),
            pl.BlockSpec((1, DIM), lambda i: (0, 0)),
            pl.BlockSpec((DIM, 6 * DIM), lambda i: (0, 0)),
            pl.BlockSpec((1, 6 * DIM), lambda i: (0, 0)),
        ],
        out_specs=pl.BlockSpec((_BN, W), lambda i: (i, 0)),
        out_shape=jax.ShapeDtypeStruct((N_PAD, W), jnp.float32),
    )(h, aggp, WrootT, bconvr, Wg, bg)


def _tc_final(hfin, batch_row, lb, W1T, W2T, boutr):
    def body(hf_ref, bt_ref, lb_ref, w1_ref, w2_ref, bo_ref, out_ref):
        lbv = lb_ref[...]
        i_ = jax.nn.sigmoid(lbv[:, 0:DIM])
        g_ = jnp.tanh(lbv[:, 2 * DIM:3 * DIM])
        o_ = jax.nn.sigmoid(lbv[:, 3 * DIM:4 * DIM])
        cl = i_ * g_
        q = o_ * jnp.tanh(cl)
        hf = hf_ref[:, 0:DIM]
        e_row = lax.dot_general(q, hf, (((1,), (1,)), ((), ())),
                                preferred_element_type=jnp.float32)
        bat = bt_ref[...]
        iota_b = lax.broadcasted_iota(jnp.int32, (B, N_PAD), 0)
        onehot = bat == iota_b
        neg = jnp.float32(-1e30)
        e_b = jnp.where(onehot, jnp.broadcast_to(e_row, (B, N_PAD)), neg)
        emax = jnp.max(e_b, axis=1, keepdims=True)
        exm = jnp.where(onehot, jnp.exp(e_b - emax), 0.0)
        denom = jnp.maximum(jnp.sum(exm, axis=1, keepdims=True), 0.5)
        alpha = exm / denom
        r_read = jnp.dot(alpha, hf, preferred_element_type=jnp.float32)
        qb = jnp.broadcast_to(q, (B, DIM))
        out_ref[...] = (jnp.dot(qb, w1_ref[...],
                                preferred_element_type=jnp.float32)
                        + jnp.dot(r_read, w2_ref[...],
                                  preferred_element_type=jnp.float32)
                        + bo_ref[...])

    return pl.pallas_call(
        body,
        grid=(1,),
        in_specs=[
            pl.BlockSpec((N_PAD, W), lambda i: (0, 0)),
            pl.BlockSpec((1, N_PAD), lambda i: (0, 0)),
            pl.BlockSpec((1, 4 * DIM), lambda i: (0, 0)),
            pl.BlockSpec((DIM, 2), lambda i: (0, 0)),
            pl.BlockSpec((DIM, 2), lambda i: (0, 0)),
            pl.BlockSpec((1, 2), lambda i: (0, 0)),
        ],
        out_specs=pl.BlockSpec((B, 2), lambda i: (0, 0)),
        out_shape=jax.ShapeDtypeStruct((B, 2), jnp.float32),
    )(hfin, batch_row, lb, W1T, W2T, boutr)


def kernel(x, edge_index, edge_attr, batch, W0, b0, We1, be1, We2, be2,
           Wroot, bconv, Wih, Whh, bih, bhh, Wlih, Wlhh, blih, blhh,
           Wout, bout):
    f32 = jnp.float32
    src = jnp.pad(edge_index[0], (0, E_PAD - E))
    dst = jnp.pad(edge_index[1], (0, E_PAD - E),
                  constant_values=N_PAD - 1)
    src_r = src.reshape(NW, NCHUNK, IR, 128)
    dst_r2 = dst.reshape(16, _NC2, IR, 128)

    ea_p = jnp.pad(edge_attr, ((0, E_PAD - E), (0, 0)))
    x_p = jnp.pad(x, ((0, N_PAD - N), (0, 0)))
    batch_row = jnp.pad(batch, (0, N_PAD - N),
                        constant_values=B)[None, :].astype(jnp.int32)

    zinit = jnp.zeros((N_PAD, W), f32)
    eye = jnp.eye(DIM, dtype=f32)
    S = jnp.repeat(eye, DIM, axis=1)
    R = jnp.tile(eye, (DIM, 1))

    wk = _tc_prep_wk(ea_p, We1.T, be1[None, :], We2.T, be2[None, :])
    h = _tc_h0(x_p, W0.T, b0[None, :])

    WrootT = Wroot.T
    bconvr = bconv[None, :]
    Wg = jnp.concatenate([Wih.T, Whh.T], axis=1)
    bg = jnp.concatenate([bih, bhh])[None, :]

    for _ in range(6):
        xj = _sc_gather(h, src_r)
        msg = _tc_msg(xj, wk, S, R)
        aggp = _sc_scatter(msg, dst_r2, zinit)
        h = _tc_gru(h, aggp, WrootT, bconvr, Wg, bg)

    lb = (blih + blhh)[None, :]
    W1T = Wout[:, 0:DIM].T
    W2T = Wout[:, DIM:2 * DIM].T
    return _tc_final(h, batch_row, lb, W1T, W2T, bout[None, :])


# speedup vs baseline: 4.2118x; 4.2118x over previous
import functools

import jax
import jax.numpy as jnp
from jax import lax
from jax.experimental import pallas as pl
from jax.experimental.pallas import tpu as pltpu
from jax.experimental.pallas import tpu_sc as plsc

N = 10000
E = 160000
NF = 8
EF = 4
DIM = 64
B = 64
W = 128

N_PAD = 10240
E_PAD = 163840
NW = 32
TE = E_PAD // NW
CH = 512
NCHUNK = TE // CH
IR = CH // 128

_mesh = functools.partial(
    plsc.VectorSubcoreMesh, core_axis_name="c", subcore_axis_name="s")


def _leaky(v):
    return jnp.where(v >= 0, v, 0.01 * v)


def _sc_gather(h, idx_r):
    @functools.partial(
        pl.kernel, mesh=_mesh(),
        out_type=jax.ShapeDtypeStruct((E_PAD, W), jnp.float32),
        scratch_types=[
            pltpu.VMEM((IR, 128), jnp.int32),
            pltpu.VMEM((CH, W), jnp.float32),
            pltpu.SemaphoreType.DMA,
        ],
    )
    def k(h_hbm, idx_hbm, out_hbm, idx_v, rows_v, sem):
        wid = lax.axis_index("s") * 2 + lax.axis_index("c")
        for c in range(NCHUNK):
            pltpu.sync_copy(idx_hbm.at[wid, c], idx_v)
            for j in range(IR):
                pltpu.async_copy(
                    h_hbm.at[idx_v.at[j]],
                    rows_v.at[pl.ds(j * 128, 128)], sem).wait()
            pltpu.sync_copy(rows_v, out_hbm.at[pl.ds(wid * TE + c * CH, CH)])

    return k(h, idx_r)


_NH = N_PAD // 2
_AR = 5248
_TRASH = 5200
_CPT = E_PAD // 16
_NC2 = _CPT // CH


def _sc_scatter(msg, dst_r2, zinit):
    @functools.partial(
        pl.kernel, mesh=_mesh(),
        out_type=jax.ShapeDtypeStruct((N_PAD, W), jnp.float32),
        scratch_types=[
            pltpu.VMEM((IR, 128), jnp.int32),
            pltpu.VMEM((CH, W), jnp.float32),
            pltpu.VMEM_SHARED((_AR, W), jnp.float32),
        ],
    )
    def k(msg_hbm, dst_hbm, z_hbm, out_hbm, idx_v, rows_v, acc_sh):
        cid = lax.axis_index("c")
        sid = lax.axis_index("s")
        lo = cid * _NH
        pltpu.sync_copy(z_hbm.at[pl.ds(sid * 328, 328)],
                        acc_sh.at[pl.ds(sid * 328, 328)])
        plsc.subcore_barrier()
        for c in range(_NC2):
            pltpu.sync_copy(dst_hbm.at[sid, c], idx_v)
            pltpu.sync_copy(
                msg_hbm.at[pl.ds(sid * _CPT + c * CH, CH)], rows_v)
            for r in range(IR):
                for v in range(8):
                    d = idx_v[r, pl.ds(v * 16, 16)] - lo
                    ok = (d >= 0) & (d < _NH)
                    idx_v[r, pl.ds(v * 16, 16)] = jnp.where(ok, d, _TRASH)
            for r in range(IR):
                pltpu.sync_copy(rows_v.at[pl.ds(r * 128, 128)],
                                acc_sh.at[idx_v.at[r]], add=True)
        plsc.subcore_barrier()
        pltpu.sync_copy(acc_sh.at[pl.ds(sid * 320, 320)],
                        out_hbm.at[pl.ds(cid * _NH + sid * 320, 320)])

    return k(msg, dst_r2, zinit)


_BE = 512


def _tc_prep_wk(edge_attr_p, We1T, be1r, We2T, be2r):
    def body(ea_ref, w1_ref, b1_ref, w2_ref, b2_ref, out_ref):
        h1 = _leaky(
            jnp.dot(ea_ref[...], w1_ref[...],
                    preferred_element_type=jnp.float32) + b1_ref[...])
        wkf = jnp.dot(
            h1, w2_ref[...], preferred_element_type=jnp.float32) + b2_ref[...]
        out_ref[...] = wkf.astype(jnp.bfloat16)

    return pl.pallas_call(
        body,
        grid=(E_PAD // _BE,),
        in_specs=[
            pl.BlockSpec((_BE, EF), lambda i: (i, 0)),
            pl.BlockSpec((EF, DIM), lambda i: (0, 0)),
            pl.BlockSpec((1, DIM), lambda i: (0, 0)),
            pl.BlockSpec((DIM, DIM * DIM), lambda i: (0, 0)),
            pl.BlockSpec((1, DIM * DIM), lambda i: (0, 0)),
        ],
        out_specs=pl.BlockSpec((_BE, DIM * DIM), lambda i: (i, 0)),
        out_shape=jax.ShapeDtypeStruct((E_PAD, DIM * DIM), jnp.bfloat16),
    )(edge_attr_p, We1T, be1r, We2T, be2r)


def _tc_msg(xj, wk, R):
    def body(xj_ref, wk_ref, r_ref, out_ref):
        x128 = xj_ref[:, 0:W]
        x128 = x128 + jnp.concatenate(
            [x128[:, DIM:W], x128[:, 0:DIM]], axis=1)
        xt = pltpu.repeat(x128, DIM // 2, axis=1)
        g = xt * wk_ref[...].astype(jnp.float32)
        acc = jnp.dot(g, r_ref[...], preferred_element_type=jnp.float32)
        one = (lax.broadcasted_iota(jnp.int32, (_BE, DIM), 1) == 0
               ).astype(jnp.float32)
        out_ref[...] = jnp.concatenate([acc, one], axis=1)

    return pl.pallas_call(
        body,
        grid=(E_PAD // _BE,),
        in_specs=[
            pl.BlockSpec((_BE, W), lambda i: (i, 0)),
            pl.BlockSpec((_BE, DIM * DIM), lambda i: (i, 0)),
            pl.BlockSpec((DIM * DIM, DIM), lambda i: (0, 0)),
        ],
        out_specs=pl.BlockSpec((_BE, W), lambda i: (i, 0)),
        out_shape=jax.ShapeDtypeStruct((E_PAD, W), jnp.float32),
    )(xj, wk, R)


def _tc_h0(x_p, W0T, b0r):
    def body(x_ref, w_ref, b_ref, out_ref):
        h0 = _leaky(
            jnp.dot(x_ref[...], w_ref[...],
                    preferred_element_type=jnp.float32) + b_ref[...])
        out_ref[...] = jnp.concatenate(
            [h0, jnp.zeros((N_PAD, W - DIM), jnp.float32)], axis=1)

    return pl.pallas_call(
        body,
        grid=(1,),
        in_specs=[
            pl.BlockSpec((N_PAD, NF), lambda i: (0, 0)),
            pl.BlockSpec((NF, DIM), lambda i: (0, 0)),
            pl.BlockSpec((1, DIM), lambda i: (0, 0)),
        ],
        out_specs=pl.BlockSpec((N_PAD, W), lambda i: (0, 0)),
        out_shape=jax.ShapeDtypeStruct((N_PAD, W), jnp.float32),
    )(x_p, W0T, b0r)


_BN = 2048


def _tc_gru(h, aggp, WrootT, bconvr, Wg, bg):
    def body(h_ref, ap_ref, wr_ref, bc_ref, wg_ref, bg_ref, out_ref):
        h_ = h_ref[:, 0:DIM]
        deg = jnp.maximum(ap_ref[:, DIM:DIM + 1], 1.0)
        agg = ap_ref[:, 0:DIM] / deg
        m = _leaky(agg + jnp.dot(h_, wr_ref[...],
                                 preferred_element_type=jnp.float32)
                   + bc_ref[...])
        gi = jnp.dot(m, wg_ref[:, 0:3 * DIM],
                     preferred_element_type=jnp.float32) + bg_ref[:, 0:3 * DIM]
        gh = jnp.dot(h_, wg_ref[:, 3 * DIM:6 * DIM],
                     preferred_element_type=jnp.float32) + bg_ref[:, 3 * DIM:]
        r = jax.nn.sigmoid(gi[:, 0:DIM] + gh[:, 0:DIM])
        z = jax.nn.sigmoid(gi[:, DIM:2 * DIM] + gh[:, DIM:2 * DIM])
        n = jnp.tanh(gi[:, 2 * DIM:3 * DIM] + r * gh[:, 2 * DIM:3 * DIM])
        hn = (1.0 - z) * n + z * h_
        out_ref[...] = jnp.concatenate(
            [hn, jnp.zeros((_BN, W - DIM), jnp.float32)], axis=1)

    return pl.pallas_call(
        body,
        grid=(N_PAD // _BN,),
        in_specs=[
            pl.BlockSpec((_BN, W), lambda i: (i, 0)),
            pl.BlockSpec((_BN, W), lambda i: (i, 0)),
            pl.BlockSpec((DIM, DIM), lambda i: (0, 0)),
            pl.BlockSpec((1, DIM), lambda i: (0, 0)),
            pl.BlockSpec((DIM, 6 * DIM), lambda i: (0, 0)),
            pl.BlockSpec((1, 6 * DIM), lambda i: (0, 0)),
        ],
        out_specs=pl.BlockSpec((_BN, W), lambda i: (i, 0)),
        out_shape=jax.ShapeDtypeStruct((N_PAD, W), jnp.float32),
    )(h, aggp, WrootT, bconvr, Wg, bg)


def _tc_final(hfin, batch_row, lb, W1T, W2T, boutr):
    def body(hf_ref, bt_ref, lb_ref, w1_ref, w2_ref, bo_ref, out_ref):
        lbv = lb_ref[...]
        i_ = jax.nn.sigmoid(lbv[:, 0:DIM])
        g_ = jnp.tanh(lbv[:, 2 * DIM:3 * DIM])
        o_ = jax.nn.sigmoid(lbv[:, 3 * DIM:4 * DIM])
        cl = i_ * g_
        q = o_ * jnp.tanh(cl)
        hf = hf_ref[:, 0:DIM]
        e_row = lax.dot_general(q, hf, (((1,), (1,)), ((), ())),
                                preferred_element_type=jnp.float32)
        bat = bt_ref[...]
        iota_b = lax.broadcasted_iota(jnp.int32, (B, N_PAD), 0)
        onehot = bat == iota_b
        neg = jnp.float32(-1e30)
        e_b = jnp.where(onehot, jnp.broadcast_to(e_row, (B, N_PAD)), neg)
        emax = jnp.max(e_b, axis=1, keepdims=True)
        exm = jnp.where(onehot, jnp.exp(e_b - emax), 0.0)
        denom = jnp.maximum(jnp.sum(exm, axis=1, keepdims=True), 0.5)
        alpha = exm / denom
        r_read = jnp.dot(alpha, hf, preferred_element_type=jnp.float32)
        qb = jnp.broadcast_to(q, (B, DIM))
        out_ref[...] = (jnp.dot(qb, w1_ref[...],
                                preferred_element_type=jnp.float32)
                        + jnp.dot(r_read, w2_ref[...],
                                  preferred_element_type=jnp.float32)
                        + bo_ref[...])

    return pl.pallas_call(
        body,
        grid=(1,),
        in_specs=[
            pl.BlockSpec((N_PAD, W), lambda i: (0, 0)),
            pl.BlockSpec((1, N_PAD), lambda i: (0, 0)),
            pl.BlockSpec((1, 4 * DIM), lambda i: (0, 0)),
            pl.BlockSpec((DIM, 2), lambda i: (0, 0)),
            pl.BlockSpec((DIM, 2), lambda i: (0, 0)),
            pl.BlockSpec((1, 2), lambda i: (0, 0)),
        ],
        out_specs=pl.BlockSpec((B, 2), lambda i: (0, 0)),
        out_shape=jax.ShapeDtypeStruct((B, 2), jnp.float32),
    )(hfin, batch_row, lb, W1T, W2T, boutr)


def kernel(x, edge_index, edge_attr, batch, W0, b0, We1, be1, We2, be2,
           Wroot, bconv, Wih, Whh, bih, bhh, Wlih, Wlhh, blih, blhh,
           Wout, bout):
    f32 = jnp.float32
    src = jnp.pad(edge_index[0], (0, E_PAD - E))
    dst = jnp.pad(edge_index[1], (0, E_PAD - E),
                  constant_values=N_PAD - 1)
    src_r = src.reshape(NW, NCHUNK, IR, 128)
    dst_r2 = dst.reshape(16, _NC2, IR, 128)

    ea_p = jnp.pad(edge_attr, ((0, E_PAD - E), (0, 0)))
    x_p = jnp.pad(x, ((0, N_PAD - N), (0, 0)))
    batch_row = jnp.pad(batch, (0, N_PAD - N),
                        constant_values=B)[None, :].astype(jnp.int32)

    zinit = jnp.zeros((N_PAD, W), f32)
    R = jnp.repeat(jnp.eye(DIM, dtype=f32), DIM, axis=0)
    We2Tp = We2.T.reshape(DIM, DIM, DIM).transpose(0, 2, 1).reshape(
        DIM, DIM * DIM)
    be2p = be2.reshape(DIM, DIM).T.reshape(DIM * DIM)

    wk = _tc_prep_wk(ea_p, We1.T, be1[None, :], We2Tp, be2p[None, :])
    h = _tc_h0(x_p, W0.T, b0[None, :])

    WrootT = Wroot.T
    bconvr = bconv[None, :]
    Wg = jnp.concatenate([Wih.T, Whh.T], axis=1)
    bg = jnp.concatenate([bih, bhh])[None, :]

    for _ in range(6):
        xj = _sc_gather(h, src_r)
        msg = _tc_msg(xj, wk, R)
        aggp = _sc_scatter(msg, dst_r2, zinit)
        h = _tc_gru(h, aggp, WrootT, bconvr, Wg, bg)

    lb = (blih + blhh)[None, :]
    W1T = Wout[:, 0:DIM].T
    W2T = Wout[:, DIM:2 * DIM].T
    return _tc_final(h, batch_row, lb, W1T, W2T, bout[None, :])


# speedup vs baseline: 4.3069x; 1.0226x over previous
import functools

import jax
import jax.numpy as jnp
from jax import lax
from jax.experimental import pallas as pl
from jax.experimental.pallas import tpu as pltpu
from jax.experimental.pallas import tpu_sc as plsc

N = 10000
E = 160000
NF = 8
EF = 4
DIM = 64
B = 64
W = 128

N_PAD = 10240
E_PAD = 163840
NW = 32
TE = E_PAD // NW
CH = 512
NCHUNK = TE // CH
IR = CH // 128

_mesh = functools.partial(
    plsc.VectorSubcoreMesh, core_axis_name="c", subcore_axis_name="s")


def _leaky(v):
    return jnp.where(v >= 0, v, 0.01 * v)


def _sc_gather(h, idx_r):
    @functools.partial(
        pl.kernel, mesh=_mesh(),
        out_type=jax.ShapeDtypeStruct((E_PAD, W), jnp.float32),
        scratch_types=[
            pltpu.VMEM((IR, 128), jnp.int32),
            pltpu.VMEM((CH, W), jnp.float32),
            pltpu.SemaphoreType.DMA,
        ],
    )
    def k(h_hbm, idx_hbm, out_hbm, idx_v, rows_v, sem):
        wid = lax.axis_index("s") * 2 + lax.axis_index("c")
        for c in range(NCHUNK):
            pltpu.sync_copy(idx_hbm.at[wid, c], idx_v)
            cps = [pltpu.async_copy(h_hbm.at[idx_v.at[j]],
                                    rows_v.at[pl.ds(j * 128, 128)], sem)
                   for j in range(IR)]
            for cp in cps:
                cp.wait()
            pltpu.sync_copy(rows_v, out_hbm.at[pl.ds(wid * TE + c * CH, CH)])

    return k(h, idx_r)


_NH = N_PAD // 2
_AR = 5248
_TRASH = 5200
_CPT = E_PAD // 16
_NC2 = _CPT // CH


def _sc_scatter(msg, dst_r2, zinit):
    @functools.partial(
        pl.kernel, mesh=_mesh(),
        out_type=jax.ShapeDtypeStruct((N_PAD, W), jnp.float32),
        scratch_types=[
            pltpu.VMEM((IR, 128), jnp.int32),
            pltpu.VMEM((CH, W), jnp.float32),
            pltpu.VMEM_SHARED((_AR, W), jnp.float32),
            pltpu.SemaphoreType.DMA,
        ],
    )
    def k(msg_hbm, dst_hbm, z_hbm, out_hbm, idx_v, rows_v, acc_sh, sem):
        cid = lax.axis_index("c")
        sid = lax.axis_index("s")
        lo = cid * _NH
        pltpu.sync_copy(z_hbm.at[pl.ds(sid * 328, 328)],
                        acc_sh.at[pl.ds(sid * 328, 328)])
        plsc.subcore_barrier()
        for c in range(_NC2):
            pltpu.sync_copy(dst_hbm.at[sid, c], idx_v)
            pltpu.sync_copy(
                msg_hbm.at[pl.ds(sid * _CPT + c * CH, CH)], rows_v)
            for r in range(IR):
                for v in range(8):
                    d = idx_v[r, pl.ds(v * 16, 16)] - lo
                    ok = (d >= 0) & (d < _NH)
                    idx_v[r, pl.ds(v * 16, 16)] = jnp.where(ok, d, _TRASH)
            cps = [pltpu.async_copy(rows_v.at[pl.ds(r * 128, 128)],
                                    acc_sh.at[idx_v.at[r]], sem,
                                    add=True)
                   for r in range(IR)]
            for cp in cps:
                cp.wait()
        plsc.subcore_barrier()
        pltpu.sync_copy(acc_sh.at[pl.ds(sid * 320, 320)],
                        out_hbm.at[pl.ds(cid * _NH + sid * 320, 320)])

    return k(msg, dst_r2, zinit)


_BE = 512


def _tc_prep_wk(edge_attr_p, We1T, be1r, We2T, be2r):
    def body(ea_ref, w1_ref, b1_ref, w2_ref, b2_ref, out_ref):
        h1 = _leaky(
            jnp.dot(ea_ref[...], w1_ref[...],
                    preferred_element_type=jnp.float32) + b1_ref[...])
        wkf = jnp.dot(
            h1, w2_ref[...], preferred_element_type=jnp.float32) + b2_ref[...]
        out_ref[...] = wkf.astype(jnp.bfloat16)

    return pl.pallas_call(
        body,
        grid=(E_PAD // _BE,),
        in_specs=[
            pl.BlockSpec((_BE, EF), lambda i: (i, 0)),
            pl.BlockSpec((EF, DIM), lambda i: (0, 0)),
            pl.BlockSpec((1, DIM), lambda i: (0, 0)),
            pl.BlockSpec((DIM, DIM * DIM), lambda i: (0, 0)),
            pl.BlockSpec((1, DIM * DIM), lambda i: (0, 0)),
        ],
        out_specs=pl.BlockSpec((_BE, DIM * DIM), lambda i: (i, 0)),
        out_shape=jax.ShapeDtypeStruct((E_PAD, DIM * DIM), jnp.bfloat16),
    )(edge_attr_p, We1T, be1r, We2T, be2r)


def _tc_msg(xj, wk, R):
    def body(xj_ref, wk_ref, r_ref, out_ref):
        x128 = xj_ref[:, 0:W]
        x128 = x128 + jnp.concatenate(
            [x128[:, DIM:W], x128[:, 0:DIM]], axis=1)
        xt = pltpu.repeat(x128, DIM // 2, axis=1)
        g = xt * wk_ref[...].astype(jnp.float32)
        acc = jnp.dot(g, r_ref[...], preferred_element_type=jnp.float32)
        one = (lax.broadcasted_iota(jnp.int32, (_BE, DIM), 1) == 0
               ).astype(jnp.float32)
        out_ref[...] = jnp.concatenate([acc, one], axis=1)

    return pl.pallas_call(
        body,
        grid=(E_PAD // _BE,),
        in_specs=[
            pl.BlockSpec((_BE, W), lambda i: (i, 0)),
            pl.BlockSpec((_BE, DIM * DIM), lambda i: (i, 0)),
            pl.BlockSpec((DIM * DIM, DIM), lambda i: (0, 0)),
        ],
        out_specs=pl.BlockSpec((_BE, W), lambda i: (i, 0)),
        out_shape=jax.ShapeDtypeStruct((E_PAD, W), jnp.float32),
    )(xj, wk, R)


def _tc_h0(x_p, W0T, b0r):
    def body(x_ref, w_ref, b_ref, out_ref):
        h0 = _leaky(
            jnp.dot(x_ref[...], w_ref[...],
                    preferred_element_type=jnp.float32) + b_ref[...])
        out_ref[...] = jnp.concatenate(
            [h0, jnp.zeros((N_PAD, W - DIM), jnp.float32)], axis=1)

    return pl.pallas_call(
        body,
        grid=(1,),
        in_specs=[
            pl.BlockSpec((N_PAD, NF), lambda i: (0, 0)),
            pl.BlockSpec((NF, DIM), lambda i: (0, 0)),
            pl.BlockSpec((1, DIM), lambda i: (0, 0)),
        ],
        out_specs=pl.BlockSpec((N_PAD, W), lambda i: (0, 0)),
        out_shape=jax.ShapeDtypeStruct((N_PAD, W), jnp.float32),
    )(x_p, W0T, b0r)


_BN = 2048


def _tc_gru(h, aggp, WrootT, bconvr, Wg, bg):
    def body(h_ref, ap_ref, wr_ref, bc_ref, wg_ref, bg_ref, out_ref):
        h_ = h_ref[:, 0:DIM]
        deg = jnp.maximum(ap_ref[:, DIM:DIM + 1], 1.0)
        agg = ap_ref[:, 0:DIM] / deg
        m = _leaky(agg + jnp.dot(h_, wr_ref[...],
                                 preferred_element_type=jnp.float32)
                   + bc_ref[...])
        gi = jnp.dot(m, wg_ref[:, 0:3 * DIM],
                     preferred_element_type=jnp.float32) + bg_ref[:, 0:3 * DIM]
        gh = jnp.dot(h_, wg_ref[:, 3 * DIM:6 * DIM],
                     preferred_element_type=jnp.float32) + bg_ref[:, 3 * DIM:]
        r = jax.nn.sigmoid(gi[:, 0:DIM] + gh[:, 0:DIM])
        z = jax.nn.sigmoid(gi[:, DIM:2 * DIM] + gh[:, DIM:2 * DIM])
        n = jnp.tanh(gi[:, 2 * DIM:3 * DIM] + r * gh[:, 2 * DIM:3 * DIM])
        hn = (1.0 - z) * n + z * h_
        out_ref[...] = jnp.concatenate(
            [hn, jnp.zeros((_BN, W - DIM), jnp.float32)], axis=1)

    return pl.pallas_call(
        body,
        grid=(N_PAD // _BN,),
        in_specs=[
            pl.BlockSpec((_BN, W), lambda i: (i, 0)),
            pl.BlockSpec((_BN, W), lambda i: (i, 0)),
            pl.BlockSpec((DIM, DIM), lambda i: (0, 0)),
            pl.BlockSpec((1, DIM), lambda i: (0, 0)),
            pl.BlockSpec((DIM, 6 * DIM), lambda i: (0, 0)),
            pl.BlockSpec((1, 6 * DIM), lambda i: (0, 0)),
        ],
        out_specs=pl.BlockSpec((_BN, W), lambda i: (i, 0)),
        out_shape=jax.ShapeDtypeStruct((N_PAD, W), jnp.float32),
    )(h, aggp, WrootT, bconvr, Wg, bg)


def _tc_final(hfin, batch_row, lb, W1T, W2T, boutr):
    def body(hf_ref, bt_ref, lb_ref, w1_ref, w2_ref, bo_ref, out_ref):
        lbv = lb_ref[...]
        i_ = jax.nn.sigmoid(lbv[:, 0:DIM])
        g_ = jnp.tanh(lbv[:, 2 * DIM:3 * DIM])
        o_ = jax.nn.sigmoid(lbv[:, 3 * DIM:4 * DIM])
        cl = i_ * g_
        q = o_ * jnp.tanh(cl)
        hf = hf_ref[:, 0:DIM]
        e_row = lax.dot_general(q, hf, (((1,), (1,)), ((), ())),
                                preferred_element_type=jnp.float32)
        bat = bt_ref[...]
        iota_b = lax.broadcasted_iota(jnp.int32, (B, N_PAD), 0)
        onehot = bat == iota_b
        neg = jnp.float32(-1e30)
        e_b = jnp.where(onehot, jnp.broadcast_to(e_row, (B, N_PAD)), neg)
        emax = jnp.max(e_b, axis=1, keepdims=True)
        exm = jnp.where(onehot, jnp.exp(e_b - emax), 0.0)
        denom = jnp.maximum(jnp.sum(exm, axis=1, keepdims=True), 0.5)
        alpha = exm / denom
        r_read = jnp.dot(alpha, hf, preferred_element_type=jnp.float32)
        qb = jnp.broadcast_to(q, (B, DIM))
        out_ref[...] = (jnp.dot(qb, w1_ref[...],
                                preferred_element_type=jnp.float32)
                        + jnp.dot(r_read, w2_ref[...],
                                  preferred_element_type=jnp.float32)
                        + bo_ref[...])

    return pl.pallas_call(
        body,
        grid=(1,),
        in_specs=[
            pl.BlockSpec((N_PAD, W), lambda i: (0, 0)),
            pl.BlockSpec((1, N_PAD), lambda i: (0, 0)),
            pl.BlockSpec((1, 4 * DIM), lambda i: (0, 0)),
            pl.BlockSpec((DIM, 2), lambda i: (0, 0)),
            pl.BlockSpec((DIM, 2), lambda i: (0, 0)),
            pl.BlockSpec((1, 2), lambda i: (0, 0)),
        ],
        out_specs=pl.BlockSpec((B, 2), lambda i: (0, 0)),
        out_shape=jax.ShapeDtypeStruct((B, 2), jnp.float32),
    )(hfin, batch_row, lb, W1T, W2T, boutr)


def kernel(x, edge_index, edge_attr, batch, W0, b0, We1, be1, We2, be2,
           Wroot, bconv, Wih, Whh, bih, bhh, Wlih, Wlhh, blih, blhh,
           Wout, bout):
    f32 = jnp.float32
    src = jnp.pad(edge_index[0], (0, E_PAD - E))
    dst = jnp.pad(edge_index[1], (0, E_PAD - E),
                  constant_values=N_PAD - 1)
    src_r = src.reshape(NW, NCHUNK, IR, 128)
    dst_r2 = dst.reshape(16, _NC2, IR, 128)

    ea_p = jnp.pad(edge_attr, ((0, E_PAD - E), (0, 0)))
    x_p = jnp.pad(x, ((0, N_PAD - N), (0, 0)))
    batch_row = jnp.pad(batch, (0, N_PAD - N),
                        constant_values=B)[None, :].astype(jnp.int32)

    zinit = jnp.zeros((N_PAD, W), f32)
    R = jnp.repeat(jnp.eye(DIM, dtype=f32), DIM, axis=0)
    We2Tp = We2.T.reshape(DIM, DIM, DIM).transpose(0, 2, 1).reshape(
        DIM, DIM * DIM)
    be2p = be2.reshape(DIM, DIM).T.reshape(DIM * DIM)

    wk = _tc_prep_wk(ea_p, We1.T, be1[None, :], We2Tp, be2p[None, :])
    h = _tc_h0(x_p, W0.T, b0[None, :])

    WrootT = Wroot.T
    bconvr = bconv[None, :]
    Wg = jnp.concatenate([Wih.T, Whh.T], axis=1)
    bg = jnp.concatenate([bih, bhh])[None, :]

    for _ in range(6):
        xj = _sc_gather(h, src_r)
        msg = _tc_msg(xj, wk, R)
        aggp = _sc_scatter(msg, dst_r2, zinit)
        h = _tc_gru(h, aggp, WrootT, bconvr, Wg, bg)

    lb = (blih + blhh)[None, :]
    W1T = Wout[:, 0:DIM].T
    W2T = Wout[:, DIM:2 * DIM].T
    return _tc_final(h, batch_row, lb, W1T, W2T, bout[None, :])


# speedup vs baseline: 4.3633x; 1.0131x over previous
import functools

import jax
import jax.numpy as jnp
from jax import lax
from jax.experimental import pallas as pl
from jax.experimental.pallas import tpu as pltpu
from jax.experimental.pallas import tpu_sc as plsc

N = 10000
E = 160000
NF = 8
EF = 4
DIM = 64
B = 64
W = 128

N_PAD = 10240
E_PAD = 163840
NW = 32
TE = E_PAD // NW
CH = 256
NCHUNK = TE // CH
IR = CH // 128

_mesh = functools.partial(
    plsc.VectorSubcoreMesh, core_axis_name="c", subcore_axis_name="s")


def _leaky(v):
    return jnp.where(v >= 0, v, 0.01 * v)


def _sc_gather(h, idx_r):
    @functools.partial(
        pl.kernel, mesh=_mesh(),
        out_type=jax.ShapeDtypeStruct((E_PAD, W), jnp.float32),
        scratch_types=[
            pltpu.VMEM((2, IR, 128), jnp.int32),
            pltpu.VMEM((2, CH, W), jnp.float32),
            pltpu.SemaphoreType.DMA,
            pltpu.SemaphoreType.DMA,
        ],
    )
    def k(h_hbm, idx_hbm, out_hbm, idx_v, rows_v, semg, semw):
        wid = lax.axis_index("s") * 2 + lax.axis_index("c")

        def idx_load(c, b):
            pltpu.sync_copy(idx_hbm.at[wid, c], idx_v.at[b])

        def fire(c, b):
            return [pltpu.async_copy(h_hbm.at[idx_v.at[b, j]],
                                     rows_v.at[b, pl.ds(j * 128, 128)], semg)
                    for j in range(IR)]

        idx_load(0, 0)
        g = {0: fire(0, 0)}
        idx_load(1, 1)
        wb = {}
        for c in range(NCHUNK):
            b = c & 1
            for d in g.pop(c):
                d.wait()
            wb[c] = pltpu.async_copy(
                rows_v.at[b], out_hbm.at[pl.ds(wid * TE + c * CH, CH)], semw)
            if c + 1 < NCHUNK:
                if c - 1 in wb:
                    wb.pop(c - 1).wait()
                g[c + 1] = fire(c + 1, 1 - b)
                if c + 2 < NCHUNK:
                    idx_load(c + 2, b)
        wb.pop(NCHUNK - 1).wait()

    return k(h, idx_r)


_NH = N_PAD // 2
_AR = 5248
_TRASH = 5200
_CPT = E_PAD // 16
_NC2 = _CPT // CH


def _sc_scatter(msg, dst2, zinit):
    @functools.partial(
        pl.kernel, mesh=_mesh(),
        out_type=jax.ShapeDtypeStruct((N_PAD, W), jnp.float32),
        scratch_types=[
            pltpu.VMEM((2, IR, 128), jnp.int32),
            pltpu.VMEM((2, CH, W), jnp.float32),
            pltpu.VMEM_SHARED((_AR, W), jnp.float32),
            pltpu.SemaphoreType.DMA,
            pltpu.SemaphoreType.DMA,
        ],
    )
    def k(msg_hbm, dst_hbm, z_hbm, out_hbm, idx_v, rows_v, acc_sh,
          semm, sema):
        cid = lax.axis_index("c")
        sid = lax.axis_index("s")
        pltpu.sync_copy(z_hbm.at[pl.ds(sid * 328, 328)],
                        acc_sh.at[pl.ds(sid * 328, 328)])
        plsc.subcore_barrier()

        def load(c, b):
            pltpu.sync_copy(dst_hbm.at[cid, sid, c], idx_v.at[b])
            return pltpu.async_copy(
                msg_hbm.at[pl.ds(sid * _CPT + c * CH, CH)],
                rows_v.at[b], semm)

        def fire_adds(c, b):
            return [pltpu.async_copy(rows_v.at[b, pl.ds(r * 128, 128)],
                                     acc_sh.at[idx_v.at[b, r]], sema,
                                     add=True)
                    for r in range(IR)]

        mload = {0: load(0, 0)}
        adds = {}
        for c in range(_NC2):
            b = c & 1
            if c - 1 in adds:
                for d in adds.pop(c - 1):
                    d.wait()
            if c + 1 < _NC2:
                mload[c + 1] = load(c + 1, 1 - b)
            mload.pop(c).wait()
            adds[c] = fire_adds(c, b)
        for d in adds.pop(_NC2 - 1):
            d.wait()
        plsc.subcore_barrier()
        pltpu.sync_copy(acc_sh.at[pl.ds(sid * 320, 320)],
                        out_hbm.at[pl.ds(cid * _NH + sid * 320, 320)])

    return k(msg, dst2, zinit)


_BE = 512


def _tc_prep_wk(edge_attr_p, We1T, be1r, We2T, be2r):
    def body(ea_ref, w1_ref, b1_ref, w2_ref, b2_ref, out_ref):
        h1 = _leaky(
            jnp.dot(ea_ref[...], w1_ref[...],
                    preferred_element_type=jnp.float32) + b1_ref[...])
        wkf = jnp.dot(
            h1, w2_ref[...], preferred_element_type=jnp.float32) + b2_ref[...]
        out_ref[...] = wkf.astype(jnp.bfloat16)

    return pl.pallas_call(
        body,
        grid=(E_PAD // _BE,),
        in_specs=[
            pl.BlockSpec((_BE, EF), lambda i: (i, 0)),
            pl.BlockSpec((EF, DIM), lambda i: (0, 0)),
            pl.BlockSpec((1, DIM), lambda i: (0, 0)),
            pl.BlockSpec((DIM, DIM * DIM), lambda i: (0, 0)),
            pl.BlockSpec((1, DIM * DIM), lambda i: (0, 0)),
        ],
        out_specs=pl.BlockSpec((_BE, DIM * DIM), lambda i: (i, 0)),
        out_shape=jax.ShapeDtypeStruct((E_PAD, DIM * DIM), jnp.bfloat16),
    )(edge_attr_p, We1T, be1r, We2T, be2r)


def _tc_msg(xj, wk, R):
    def body(xj_ref, wk_ref, r_ref, out_ref):
        x128 = xj_ref[:, 0:W]
        x128 = x128 + jnp.concatenate(
            [x128[:, DIM:W], x128[:, 0:DIM]], axis=1)
        xt = pltpu.repeat(x128, DIM // 2, axis=1)
        g = xt * wk_ref[...].astype(jnp.float32)
        acc = jnp.dot(g, r_ref[...], preferred_element_type=jnp.float32)
        one = (lax.broadcasted_iota(jnp.int32, (_BE, DIM), 1) == 0
               ).astype(jnp.float32)
        out_ref[...] = jnp.concatenate([acc, one], axis=1)

    return pl.pallas_call(
        body,
        grid=(E_PAD // _BE,),
        in_specs=[
            pl.BlockSpec((_BE, W), lambda i: (i, 0)),
            pl.BlockSpec((_BE, DIM * DIM), lambda i: (i, 0)),
            pl.BlockSpec((DIM * DIM, DIM), lambda i: (0, 0)),
        ],
        out_specs=pl.BlockSpec((_BE, W), lambda i: (i, 0)),
        out_shape=jax.ShapeDtypeStruct((E_PAD, W), jnp.float32),
    )(xj, wk, R)


def _tc_h0(x_p, W0T, b0r):
    def body(x_ref, w_ref, b_ref, out_ref):
        h0 = _leaky(
            jnp.dot(x_ref[...], w_ref[...],
                    preferred_element_type=jnp.float32) + b_ref[...])
        out_ref[...] = jnp.concatenate(
            [h0, jnp.zeros((N_PAD, W - DIM), jnp.float32)], axis=1)

    return pl.pallas_call(
        body,
        grid=(1,),
        in_specs=[
            pl.BlockSpec((N_PAD, NF), lambda i: (0, 0)),
            pl.BlockSpec((NF, DIM), lambda i: (0, 0)),
            pl.BlockSpec((1, DIM), lambda i: (0, 0)),
        ],
        out_specs=pl.BlockSpec((N_PAD, W), lambda i: (0, 0)),
        out_shape=jax.ShapeDtypeStruct((N_PAD, W), jnp.float32),
    )(x_p, W0T, b0r)


_BN = 2048


def _tc_gru(h, aggp, WrootT, bconvr, Wg, bg):
    def body(h_ref, ap_ref, wr_ref, bc_ref, wg_ref, bg_ref, out_ref):
        h_ = h_ref[:, 0:DIM]
        deg = jnp.maximum(ap_ref[:, DIM:DIM + 1], 1.0)
        agg = ap_ref[:, 0:DIM] / deg
        m = _leaky(agg + jnp.dot(h_, wr_ref[...],
                                 preferred_element_type=jnp.float32)
                   + bc_ref[...])
        gi = jnp.dot(m, wg_ref[:, 0:3 * DIM],
                     preferred_element_type=jnp.float32) + bg_ref[:, 0:3 * DIM]
        gh = jnp.dot(h_, wg_ref[:, 3 * DIM:6 * DIM],
                     preferred_element_type=jnp.float32) + bg_ref[:, 3 * DIM:]
        r = jax.nn.sigmoid(gi[:, 0:DIM] + gh[:, 0:DIM])
        z = jax.nn.sigmoid(gi[:, DIM:2 * DIM] + gh[:, DIM:2 * DIM])
        n = jnp.tanh(gi[:, 2 * DIM:3 * DIM] + r * gh[:, 2 * DIM:3 * DIM])
        hn = (1.0 - z) * n + z * h_
        out_ref[...] = jnp.concatenate(
            [hn, jnp.zeros((_BN, W - DIM), jnp.float32)], axis=1)

    return pl.pallas_call(
        body,
        grid=(N_PAD // _BN,),
        in_specs=[
            pl.BlockSpec((_BN, W), lambda i: (i, 0)),
            pl.BlockSpec((_BN, W), lambda i: (i, 0)),
            pl.BlockSpec((DIM, DIM), lambda i: (0, 0)),
            pl.BlockSpec((1, DIM), lambda i: (0, 0)),
            pl.BlockSpec((DIM, 6 * DIM), lambda i: (0, 0)),
            pl.BlockSpec((1, 6 * DIM), lambda i: (0, 0)),
        ],
        out_specs=pl.BlockSpec((_BN, W), lambda i: (i, 0)),
        out_shape=jax.ShapeDtypeStruct((N_PAD, W), jnp.float32),
    )(h, aggp, WrootT, bconvr, Wg, bg)


def _tc_final(hfin, batch_row, lb, W1T, W2T, boutr):
    def body(hf_ref, bt_ref, lb_ref, w1_ref, w2_ref, bo_ref, out_ref):
        lbv = lb_ref[...]
        i_ = jax.nn.sigmoid(lbv[:, 0:DIM])
        g_ = jnp.tanh(lbv[:, 2 * DIM:3 * DIM])
        o_ = jax.nn.sigmoid(lbv[:, 3 * DIM:4 * DIM])
        cl = i_ * g_
        q = o_ * jnp.tanh(cl)
        hf = hf_ref[:, 0:DIM]
        e_row = lax.dot_general(q, hf, (((1,), (1,)), ((), ())),
                                preferred_element_type=jnp.float32)
        bat = bt_ref[...]
        iota_b = lax.broadcasted_iota(jnp.int32, (B, N_PAD), 0)
        onehot = bat == iota_b
        neg = jnp.float32(-1e30)
        e_b = jnp.where(onehot, jnp.broadcast_to(e_row, (B, N_PAD)), neg)
        emax = jnp.max(e_b, axis=1, keepdims=True)
        exm = jnp.where(onehot, jnp.exp(e_b - emax), 0.0)
        denom = jnp.maximum(jnp.sum(exm, axis=1, keepdims=True), 0.5)
        alpha = exm / denom
        r_read = jnp.dot(alpha, hf, preferred_element_type=jnp.float32)
        qb = jnp.broadcast_to(q, (B, DIM))
        out_ref[...] = (jnp.dot(qb, w1_ref[...],
                                preferred_element_type=jnp.float32)
                        + jnp.dot(r_read, w2_ref[...],
                                  preferred_element_type=jnp.float32)
                        + bo_ref[...])

    return pl.pallas_call(
        body,
        grid=(1,),
        in_specs=[
            pl.BlockSpec((N_PAD, W), lambda i: (0, 0)),
            pl.BlockSpec((1, N_PAD), lambda i: (0, 0)),
            pl.BlockSpec((1, 4 * DIM), lambda i: (0, 0)),
            pl.BlockSpec((DIM, 2), lambda i: (0, 0)),
            pl.BlockSpec((DIM, 2), lambda i: (0, 0)),
            pl.BlockSpec((1, 2), lambda i: (0, 0)),
        ],
        out_specs=pl.BlockSpec((B, 2), lambda i: (0, 0)),
        out_shape=jax.ShapeDtypeStruct((B, 2), jnp.float32),
    )(hfin, batch_row, lb, W1T, W2T, boutr)


def kernel(x, edge_index, edge_attr, batch, W0, b0, We1, be1, We2, be2,
           Wroot, bconv, Wih, Whh, bih, bhh, Wlih, Wlhh, blih, blhh,
           Wout, bout):
    f32 = jnp.float32
    src = jnp.pad(edge_index[0], (0, E_PAD - E))
    dst = jnp.pad(edge_index[1], (0, E_PAD - E),
                  constant_values=N_PAD - 1)
    src_r = src.reshape(NW, NCHUNK, IR, 128)
    dstA = jnp.where(dst < _NH, dst, _TRASH)
    dstB = jnp.where(dst >= _NH, dst - _NH, _TRASH)
    dst2 = jnp.stack([dstA, dstB]).reshape(2, 16, _NC2, IR, 128)

    ea_p = jnp.pad(edge_attr, ((0, E_PAD - E), (0, 0)))
    x_p = jnp.pad(x, ((0, N_PAD - N), (0, 0)))
    batch_row = jnp.pad(batch, (0, N_PAD - N),
                        constant_values=B)[None, :].astype(jnp.int32)

    zinit = jnp.zeros((N_PAD, W), f32)
    R = jnp.repeat(jnp.eye(DIM, dtype=f32), DIM, axis=0)
    We2Tp = We2.T.reshape(DIM, DIM, DIM).transpose(0, 2, 1).reshape(
        DIM, DIM * DIM)
    be2p = be2.reshape(DIM, DIM).T.reshape(DIM * DIM)

    wk = _tc_prep_wk(ea_p, We1.T, be1[None, :], We2Tp, be2p[None, :])
    h = _tc_h0(x_p, W0.T, b0[None, :])

    WrootT = Wroot.T
    bconvr = bconv[None, :]
    Wg = jnp.concatenate([Wih.T, Whh.T], axis=1)
    bg = jnp.concatenate([bih, bhh])[None, :]

    for _ in range(6):
        xj = _sc_gather(h, src_r)
        msg = _tc_msg(xj, wk, R)
        aggp = _sc_scatter(msg, dst2, zinit)
        h = _tc_gru(h, aggp, WrootT, bconvr, Wg, bg)

    lb = (blih + blhh)[None, :]
    W1T = Wout[:, 0:DIM].T
    W2T = Wout[:, DIM:2 * DIM].T
    return _tc_final(h, batch_row, lb, W1T, W2T, bout[None, :])
